# baseline (speedup 1.0000x reference)
import functools

import jax
import jax.numpy as jnp
from jax.experimental import pallas as pl
from jax.experimental.pallas import tpu as pltpu

EPS = 1e-5


def _bn(x, g, b):
    axes = tuple(range(x.ndim - 1))
    mean = jnp.mean(x, axis=axes, keepdims=True)
    var = jnp.var(x, axis=axes, keepdims=True)
    return g * (x - mean) / jnp.sqrt(var + EPS) + b


def _linear(x, W, b=None):
    y = x @ W.T
    if b is not None:
        y = y + b
    return y


def _knn_idx(pq, pr, k):
    d = jnp.sum((pq[:, None, :] - pr[None, :, :]) ** 2, axis=-1)
    _, idx = jax.lax.top_k(-d, k)
    return idx


def _query_and_group(nsample, p_ref, p_query, feat, use_xyz):
    def per_batch(pr, pq, f):
        idx = _knn_idx(pq, pr, nsample)
        g_xyz = pr[idx] - pq[:, None, :]
        g_feat = f[idx]
        if use_xyz:
            return jnp.concatenate([g_xyz, g_feat], axis=-1)
        return g_feat
    return jax.vmap(per_batch)(p_ref, p_query, feat)


def _fps(p, m):
    n = p.shape[0]
    def body(i, state):
        idxs, dists, last = state
        d = jnp.sum((p - p[last]) ** 2, axis=-1)
        dists = jnp.minimum(dists, d)
        nxt = jnp.argmax(dists).astype(jnp.int32)
        idxs = idxs.at[i].set(nxt)
        return (idxs, dists, nxt)
    idxs = jnp.zeros((m,), jnp.int32)
    dists = jnp.full((n,), jnp.inf, jnp.float32)
    idxs, dists, last = jax.lax.fori_loop(1, m, body, (idxs, dists, jnp.int32(0)))
    return idxs


def _pt_layer(p, x, prm, nsample, share):
    xq = _linear(x, prm['q_w'], prm['q_b'])
    xk = _linear(x, prm['k_w'], prm['k_b'])
    xv = _linear(x, prm['v_w'], prm['v_b'])
    gk = _query_and_group(nsample, p, p, xk, True)
    gv = _query_and_group(nsample, p, p, xv, False)
    p_r, xk_g = gk[..., :3], gk[..., 3:]
    p_r = _linear(p_r, prm['p0_w'], prm['p0_b'])
    p_r = _bn(p_r, prm['p1_g'], prm['p1_b'])
    p_r = jax.nn.relu(p_r)
    p_r = _linear(p_r, prm['p3_w'], prm['p3_b'])
    w = xk_g - xq[:, :, None, :] + p_r
    w = _bn(w, prm['w0_g'], prm['w0_b'])
    w = jax.nn.relu(w)
    w = _linear(w, prm['w2_w'], prm['w2_b'])
    w = _bn(w, prm['w3_g'], prm['w3_b'])
    w = jax.nn.relu(w)
    w = _linear(w, prm['w5_w'], prm['w5_b'])
    w = jax.nn.softmax(w, axis=2)
    B, n, ns, c = gv.shape
    s = share
    out = ((gv + p_r).reshape(B, n, ns, s, c // s) * w[:, :, :, None, :]).sum(axis=2).reshape(B, n, c)
    return out


def _transition_down(p, x, prm, stride, nsample):
    if stride == 1:
        x = jax.nn.relu(_bn(_linear(x, prm['lin_w']), prm['bn_g'], prm['bn_b']))
        return p, x
    B, n, _ = p.shape
    m = n // stride
    idx = jax.vmap(lambda pp: _fps(pp, m))(p)
    n_p = jnp.take_along_axis(p, idx[:, :, None].astype(jnp.int32), axis=1)
    g = _query_and_group(nsample, p, n_p, x, True)
    y = _linear(g, prm['lin_w'])
    y = _bn(y, prm['bn_g'], prm['bn_b'])
    y = jax.nn.relu(y)
    y = jnp.max(y, axis=2)
    return n_p, y


def _pt_block(p, x, prm, nsample, share):
    identity = x
    x = jax.nn.relu(_bn(_linear(x, prm['l1_w']), prm['bn1_g'], prm['bn1_b']))
    x = jax.nn.relu(_bn(_pt_layer(p, x, prm['tr'], nsample, share), prm['bn2_g'], prm['bn2_b']))
    x = _bn(_linear(x, prm['l3_w']), prm['bn3_g'], prm['bn3_b'])
    x = jax.nn.relu(x + identity)
    return x


def _head_body(x_ref, w0_ref, b0_ref, w3_ref, b3_ref, out_ref):
    x = x_ref[...]
    g = jnp.mean(x, axis=1)
    h = g @ w0_ref[...].T + b0_ref[...]
    mean = jnp.mean(h, axis=0, keepdims=True)
    var = jnp.mean((h - mean) ** 2, axis=0, keepdims=True)
    h = (h - mean) / jnp.sqrt(var + EPS)
    h = jnp.maximum(h, 0.0)
    out_ref[...] = h @ w3_ref[...].T + b3_ref[...]


def _head(x, params):
    B, n, c = x.shape
    return pl.pallas_call(
        _head_body,
        out_shape=jax.ShapeDtypeStruct((B, c), jnp.float32),
    )(x, params['head0_w'], params['head0_b'], params['head3_w'], params['head3_b'])


def kernel(x_bcn, params):
    pts = jnp.transpose(x_bcn, (0, 2, 1))
    p = pts[:, :, :3]
    x = pts
    specs = [(1, 8), (4, 16), (4, 16), (4, 16)]
    for i, (stride, ns) in enumerate(specs):
        ep = params['enc%d' % (i + 1)]
        p, x = _transition_down(p, x, ep['td'], stride, ns)
        x = _pt_block(p, x, ep['blk'], ns, 8)
    return _head(x, params)


# speedup vs baseline: 1.2780x; 1.2780x over previous
import functools

import jax
import jax.numpy as jnp
from jax.experimental import pallas as pl
from jax.experimental.pallas import tpu as pltpu

EPS = 1e-5


def _bn(x, g, b):
    axes = tuple(range(x.ndim - 1))
    mean = jnp.mean(x, axis=axes, keepdims=True)
    var = jnp.var(x, axis=axes, keepdims=True)
    return g * (x - mean) / jnp.sqrt(var + EPS) + b


def _linear(x, W, b=None):
    y = x @ W.T
    if b is not None:
        y = y + b
    return y


def _knn_idx(pq, pr, k):
    d = jnp.sum((pq[:, None, :] - pr[None, :, :]) ** 2, axis=-1)
    _, idx = jax.lax.top_k(-d, k)
    return idx


def _query_and_group(nsample, p_ref, p_query, feat, use_xyz):
    def per_batch(pr, pq, f):
        idx = _knn_idx(pq, pr, nsample)
        g_xyz = pr[idx] - pq[:, None, :]
        g_feat = f[idx]
        if use_xyz:
            return jnp.concatenate([g_xyz, g_feat], axis=-1)
        return g_feat
    return jax.vmap(per_batch)(p_ref, p_query, feat)


def _fps_body(m, p3_ref, idx_ref, np_ref, dists_ref):
    B, _, _, n8 = p3_ref.shape
    n = 8 * n8
    p3 = p3_ref[...]
    pos = (jax.lax.broadcasted_iota(jnp.int32, (B, 8, n8), 1) * n8
           + jax.lax.broadcasted_iota(jnp.int32, (B, 8, n8), 2))
    lane_m = jax.lax.broadcasted_iota(jnp.int32, (B, m), 1)
    lane_3m = jax.lax.broadcasted_iota(jnp.int32, (B, 3, m), 2)

    idx_ref[...] = jnp.zeros((B, m), jnp.int32)
    np_ref[...] = jnp.zeros((B, 3, m), jnp.float32)
    dists_ref[...] = jnp.full((B, 8, n8), jnp.inf, jnp.float32)

    def _rmax(a):
        return a.max(axis=2, keepdims=True).max(axis=1, keepdims=True)

    def _rmin(a):
        return a.min(axis=2, keepdims=True).min(axis=1, keepdims=True)

    def _rsum(a):
        return a.sum(axis=2, keepdims=True).sum(axis=1, keepdims=True)

    def _coords(last):
        sel = (pos == last).astype(jnp.float32)
        px = _rsum(p3[:, 0] * sel)
        py = _rsum(p3[:, 1] * sel)
        pz = _rsum(p3[:, 2] * sel)
        return px, py, pz

    def body(i, last):
        px, py, pz = _coords(last)
        d = (p3[:, 0] - px) ** 2 + (p3[:, 1] - py) ** 2 + (p3[:, 2] - pz) ** 2
        dists = jnp.minimum(dists_ref[...], d)
        dists_ref[...] = dists
        mx = _rmax(dists)
        nxt = _rmin(jnp.where(dists == mx, pos, n))
        idx_ref[...] = jnp.where(lane_m == i, nxt[:, :, 0], idx_ref[...])
        coords = jnp.concatenate([px, py, pz], axis=1)
        np_ref[...] = jnp.where(lane_3m == i - 1, coords, np_ref[...])
        return nxt

    last = jax.lax.fori_loop(1, m, body, jnp.zeros((B, 1, 1), jnp.int32))
    px, py, pz = _coords(last)
    coords = jnp.concatenate([px, py, pz], axis=1)
    np_ref[...] = jnp.where(lane_3m == m - 1, coords, np_ref[...])


def _fps_pallas(p, m):
    B, n, _ = p.shape
    n8 = n // 8
    p3 = jnp.transpose(p, (0, 2, 1)).reshape(B, 3, 8, n8)
    idx, np_ = pl.pallas_call(
        functools.partial(_fps_body, m),
        out_shape=(jax.ShapeDtypeStruct((B, m), jnp.int32),
                   jax.ShapeDtypeStruct((B, 3, m), jnp.float32)),
        scratch_shapes=[pltpu.VMEM((B, 8, n8), jnp.float32)],
    )(p3)
    return idx, jnp.transpose(np_, (0, 2, 1))


def _pt_layer(p, x, prm, nsample, share):
    xq = _linear(x, prm['q_w'], prm['q_b'])
    xk = _linear(x, prm['k_w'], prm['k_b'])
    xv = _linear(x, prm['v_w'], prm['v_b'])
    gk = _query_and_group(nsample, p, p, xk, True)
    gv = _query_and_group(nsample, p, p, xv, False)
    p_r, xk_g = gk[..., :3], gk[..., 3:]
    p_r = _linear(p_r, prm['p0_w'], prm['p0_b'])
    p_r = _bn(p_r, prm['p1_g'], prm['p1_b'])
    p_r = jax.nn.relu(p_r)
    p_r = _linear(p_r, prm['p3_w'], prm['p3_b'])
    w = xk_g - xq[:, :, None, :] + p_r
    w = _bn(w, prm['w0_g'], prm['w0_b'])
    w = jax.nn.relu(w)
    w = _linear(w, prm['w2_w'], prm['w2_b'])
    w = _bn(w, prm['w3_g'], prm['w3_b'])
    w = jax.nn.relu(w)
    w = _linear(w, prm['w5_w'], prm['w5_b'])
    w = jax.nn.softmax(w, axis=2)
    B, n, ns, c = gv.shape
    s = share
    out = ((gv + p_r).reshape(B, n, ns, s, c // s) * w[:, :, :, None, :]).sum(axis=2).reshape(B, n, c)
    return out


def _transition_down(p, x, prm, stride, nsample):
    if stride == 1:
        x = jax.nn.relu(_bn(_linear(x, prm['lin_w']), prm['bn_g'], prm['bn_b']))
        return p, x
    B, n, _ = p.shape
    m = n // stride
    idx, n_p = _fps_pallas(p, m)
    g = _query_and_group(nsample, p, n_p, x, True)
    y = _linear(g, prm['lin_w'])
    y = _bn(y, prm['bn_g'], prm['bn_b'])
    y = jax.nn.relu(y)
    y = jnp.max(y, axis=2)
    return n_p, y


def _pt_block(p, x, prm, nsample, share):
    identity = x
    x = jax.nn.relu(_bn(_linear(x, prm['l1_w']), prm['bn1_g'], prm['bn1_b']))
    x = jax.nn.relu(_bn(_pt_layer(p, x, prm['tr'], nsample, share), prm['bn2_g'], prm['bn2_b']))
    x = _bn(_linear(x, prm['l3_w']), prm['bn3_g'], prm['bn3_b'])
    x = jax.nn.relu(x + identity)
    return x


def _head_body(x_ref, w0_ref, b0_ref, w3_ref, b3_ref, out_ref):
    x = x_ref[...]
    g = jnp.mean(x, axis=1)
    h = g @ w0_ref[...].T + b0_ref[...]
    mean = jnp.mean(h, axis=0, keepdims=True)
    var = jnp.mean((h - mean) ** 2, axis=0, keepdims=True)
    h = (h - mean) / jnp.sqrt(var + EPS)
    h = jnp.maximum(h, 0.0)
    out_ref[...] = h @ w3_ref[...].T + b3_ref[...]


def _head(x, params):
    B, n, c = x.shape
    return pl.pallas_call(
        _head_body,
        out_shape=jax.ShapeDtypeStruct((B, c), jnp.float32),
    )(x, params['head0_w'], params['head0_b'], params['head3_w'], params['head3_b'])


def kernel(x_bcn, params):
    pts = jnp.transpose(x_bcn, (0, 2, 1))
    p = pts[:, :, :3]
    x = pts
    specs = [(1, 8), (4, 16), (4, 16), (4, 16)]
    for i, (stride, ns) in enumerate(specs):
        ep = params['enc%d' % (i + 1)]
        p, x = _transition_down(p, x, ep['td'], stride, ns)
        x = _pt_block(p, x, ep['blk'], ns, 8)
    return _head(x, params)


# speedup vs baseline: 2.9026x; 2.2712x over previous
import functools

import jax
import jax.numpy as jnp
from jax.experimental import pallas as pl
from jax.experimental.pallas import tpu as pltpu

EPS = 1e-5


def _bn(x, g, b):
    axes = tuple(range(x.ndim - 1))
    mean = jnp.mean(x, axis=axes, keepdims=True)
    var = jnp.var(x, axis=axes, keepdims=True)
    return g * (x - mean) / jnp.sqrt(var + EPS) + b


def _linear(x, W, b=None):
    y = x @ W.T
    if b is not None:
        y = y + b
    return y


def _knn_body(k, pq_ref, pr_ref, idx_ref):
    _, Q, _ = pq_ref.shape
    _, _, nr = pr_ref.shape
    qx = pq_ref[0][:, 0:1]
    qy = pq_ref[0][:, 1:2]
    qz = pq_ref[0][:, 2:3]
    rx = pr_ref[0][0:1, :]
    ry = pr_ref[0][1:2, :]
    rz = pr_ref[0][2:3, :]
    d = (qx - rx) ** 2 + (qy - ry) ** 2 + (qz - rz) ** 2
    lane = jax.lax.broadcasted_iota(jnp.int32, (Q, nr), 1)
    js = []
    for _ in range(k):
        mn = jnp.min(d, axis=1, keepdims=True)
        j = jnp.min(jnp.where(d == mn, lane, nr), axis=1, keepdims=True)
        js.append(j)
        d = jnp.where(lane == j, jnp.inf, d)
    idx_ref[0] = jnp.concatenate(js, axis=1)


def _knn_pallas(pq, pr, k):
    B, nq, _ = pq.shape
    _, nr, _ = pr.shape
    Q = min(nq, 512)
    pr3 = jnp.transpose(pr, (0, 2, 1))
    return pl.pallas_call(
        functools.partial(_knn_body, k),
        grid=(B, nq // Q),
        in_specs=[
            pl.BlockSpec((1, Q, 3), lambda b, q: (b, q, 0)),
            pl.BlockSpec((1, 3, nr), lambda b, q: (b, 0, 0)),
        ],
        out_specs=pl.BlockSpec((1, Q, k), lambda b, q: (b, q, 0)),
        out_shape=jax.ShapeDtypeStruct((B, nq, k), jnp.int32),
    )(pq, pr3)


def _query_and_group_idx(idx, p_ref, p_query, feat, use_xyz):
    def per_batch(pr, pq, f, ib):
        g_xyz = pr[ib] - pq[:, None, :]
        g_feat = f[ib]
        if use_xyz:
            return jnp.concatenate([g_xyz, g_feat], axis=-1)
        return g_feat
    return jax.vmap(per_batch)(p_ref, p_query, feat, idx)


def _query_and_group(nsample, p_ref, p_query, feat, use_xyz):
    idx = _knn_pallas(p_query, p_ref, nsample)
    return _query_and_group_idx(idx, p_ref, p_query, feat, use_xyz)


def _fps_body(m, p3_ref, idx_ref, np_ref, dists_ref):
    B, _, _, n8 = p3_ref.shape
    n = 8 * n8
    p3 = p3_ref[...]
    pos = (jax.lax.broadcasted_iota(jnp.int32, (B, 8, n8), 1) * n8
           + jax.lax.broadcasted_iota(jnp.int32, (B, 8, n8), 2))
    lane_m = jax.lax.broadcasted_iota(jnp.int32, (B, m), 1)
    lane_3m = jax.lax.broadcasted_iota(jnp.int32, (B, 3, m), 2)

    idx_ref[...] = jnp.zeros((B, m), jnp.int32)
    np_ref[...] = jnp.zeros((B, 3, m), jnp.float32)
    dists_ref[...] = jnp.full((B, 8, n8), jnp.inf, jnp.float32)

    def _rmax(a):
        return a.max(axis=2, keepdims=True).max(axis=1, keepdims=True)

    def _rmin(a):
        return a.min(axis=2, keepdims=True).min(axis=1, keepdims=True)

    def _rsum(a):
        return a.sum(axis=2, keepdims=True).sum(axis=1, keepdims=True)

    def _coords(last):
        sel = (pos == last).astype(jnp.float32)
        px = _rsum(p3[:, 0] * sel)
        py = _rsum(p3[:, 1] * sel)
        pz = _rsum(p3[:, 2] * sel)
        return px, py, pz

    def body(i, last):
        px, py, pz = _coords(last)
        d = (p3[:, 0] - px) ** 2 + (p3[:, 1] - py) ** 2 + (p3[:, 2] - pz) ** 2
        dists = jnp.minimum(dists_ref[...], d)
        dists_ref[...] = dists
        mx = _rmax(dists)
        nxt = _rmin(jnp.where(dists == mx, pos, n))
        idx_ref[...] = jnp.where(lane_m == i, nxt[:, :, 0], idx_ref[...])
        coords = jnp.concatenate([px, py, pz], axis=1)
        np_ref[...] = jnp.where(lane_3m == i - 1, coords, np_ref[...])
        return nxt

    last = jax.lax.fori_loop(1, m, body, jnp.zeros((B, 1, 1), jnp.int32))
    px, py, pz = _coords(last)
    coords = jnp.concatenate([px, py, pz], axis=1)
    np_ref[...] = jnp.where(lane_3m == m - 1, coords, np_ref[...])


def _fps_pallas(p, m):
    B, n, _ = p.shape
    n8 = n // 8
    p3 = jnp.transpose(p, (0, 2, 1)).reshape(B, 3, 8, n8)
    idx, np_ = pl.pallas_call(
        functools.partial(_fps_body, m),
        out_shape=(jax.ShapeDtypeStruct((B, m), jnp.int32),
                   jax.ShapeDtypeStruct((B, 3, m), jnp.float32)),
        scratch_shapes=[pltpu.VMEM((B, 8, n8), jnp.float32)],
    )(p3)
    return idx, jnp.transpose(np_, (0, 2, 1))


def _pt_layer(p, x, prm, nsample, share):
    xq = _linear(x, prm['q_w'], prm['q_b'])
    xk = _linear(x, prm['k_w'], prm['k_b'])
    xv = _linear(x, prm['v_w'], prm['v_b'])
    gk = _query_and_group(nsample, p, p, xk, True)
    gv = _query_and_group(nsample, p, p, xv, False)
    p_r, xk_g = gk[..., :3], gk[..., 3:]
    p_r = _linear(p_r, prm['p0_w'], prm['p0_b'])
    p_r = _bn(p_r, prm['p1_g'], prm['p1_b'])
    p_r = jax.nn.relu(p_r)
    p_r = _linear(p_r, prm['p3_w'], prm['p3_b'])
    w = xk_g - xq[:, :, None, :] + p_r
    w = _bn(w, prm['w0_g'], prm['w0_b'])
    w = jax.nn.relu(w)
    w = _linear(w, prm['w2_w'], prm['w2_b'])
    w = _bn(w, prm['w3_g'], prm['w3_b'])
    w = jax.nn.relu(w)
    w = _linear(w, prm['w5_w'], prm['w5_b'])
    w = jax.nn.softmax(w, axis=2)
    B, n, ns, c = gv.shape
    s = share
    out = ((gv + p_r).reshape(B, n, ns, s, c // s) * w[:, :, :, None, :]).sum(axis=2).reshape(B, n, c)
    return out


def _transition_down(p, x, prm, stride, nsample):
    if stride == 1:
        x = jax.nn.relu(_bn(_linear(x, prm['lin_w']), prm['bn_g'], prm['bn_b']))
        return p, x
    B, n, _ = p.shape
    m = n // stride
    idx, n_p = _fps_pallas(p, m)
    g = _query_and_group(nsample, p, n_p, x, True)
    y = _linear(g, prm['lin_w'])
    y = _bn(y, prm['bn_g'], prm['bn_b'])
    y = jax.nn.relu(y)
    y = jnp.max(y, axis=2)
    return n_p, y


def _pt_block(p, x, prm, nsample, share):
    identity = x
    x = jax.nn.relu(_bn(_linear(x, prm['l1_w']), prm['bn1_g'], prm['bn1_b']))
    x = jax.nn.relu(_bn(_pt_layer(p, x, prm['tr'], nsample, share), prm['bn2_g'], prm['bn2_b']))
    x = _bn(_linear(x, prm['l3_w']), prm['bn3_g'], prm['bn3_b'])
    x = jax.nn.relu(x + identity)
    return x


def _head_body(x_ref, w0_ref, b0_ref, w3_ref, b3_ref, out_ref):
    x = x_ref[...]
    g = jnp.mean(x, axis=1)
    h = g @ w0_ref[...].T + b0_ref[...]
    mean = jnp.mean(h, axis=0, keepdims=True)
    var = jnp.mean((h - mean) ** 2, axis=0, keepdims=True)
    h = (h - mean) / jnp.sqrt(var + EPS)
    h = jnp.maximum(h, 0.0)
    out_ref[...] = h @ w3_ref[...].T + b3_ref[...]


def _head(x, params):
    B, n, c = x.shape
    return pl.pallas_call(
        _head_body,
        out_shape=jax.ShapeDtypeStruct((B, c), jnp.float32),
    )(x, params['head0_w'], params['head0_b'], params['head3_w'], params['head3_b'])


def kernel(x_bcn, params):
    pts = jnp.transpose(x_bcn, (0, 2, 1))
    p = pts[:, :, :3]
    x = pts
    specs = [(1, 8), (4, 16), (4, 16), (4, 16)]
    for i, (stride, ns) in enumerate(specs):
        ep = params['enc%d' % (i + 1)]
        p, x = _transition_down(p, x, ep['td'], stride, ns)
        x = _pt_block(p, x, ep['blk'], ns, 8)
    return _head(x, params)


# speedup vs baseline: 15.4809x; 5.3335x over previous
import functools

import jax
import jax.numpy as jnp
from jax import lax
from jax.experimental import pallas as pl
from jax.experimental.pallas import tpu as pltpu
from jax.experimental.pallas import tpu_sc as plsc

EPS = 1e-5

_SC_INFO = plsc.get_sparse_core_info()
_NC, _NS, _L = _SC_INFO.num_cores, _SC_INFO.num_subcores, _SC_INFO.num_lanes
_NW = _NC * _NS


def _sc_gather_body(n_table, rows_pw, C, wpb, table_hbm, idx_hbm, out_hbm,
                    idx_v, rows_v, sem):
    wid = lax.axis_index("s") * _NC + lax.axis_index("c")
    base_row = wid * rows_pw
    badd = (wid // wpb) * n_table

    def chunk(t, _):
        off = base_row + t * C
        pltpu.sync_copy(idx_hbm.at[pl.ds(off, C)], idx_v)
        for i in range(C // _L):
            sl = pl.ds(i * _L, _L)
            idx_v[sl] = idx_v[sl] + badd
        pltpu.async_copy(table_hbm.at[idx_v], rows_v, sem).wait()
        pltpu.sync_copy(rows_v, out_hbm.at[pl.ds(off, C)])
        return _

    lax.fori_loop(0, rows_pw // C, chunk, 0)


def _sc_gather(table, idx):
    B, n, D = table.shape
    _, nq, ns = idx.shape
    nidx = B * nq * ns
    assert D % _L == 0 and nidx % (8 * _NW) == 0 and _NW % B == 0
    rows_pw = nidx // _NW
    wpb = _NW // B
    C = 128
    while C * D * 4 > 380 * 1024 or rows_pw % C:
        C //= 2
    assert C >= 8 and C % 8 == 0

    mesh = plsc.VectorSubcoreMesh(core_axis_name="c", subcore_axis_name="s")
    out = pl.kernel(
        functools.partial(_sc_gather_body, n, rows_pw, C, wpb),
        mesh=mesh,
        compiler_params=pltpu.CompilerParams(use_tc_tiling_on_sc=False),
        out_type=jax.ShapeDtypeStruct((nidx, D), jnp.float32),
        scratch_types=[
            pltpu.VMEM((C,), jnp.int32),
            pltpu.VMEM((C, D), jnp.float32),
            pltpu.SemaphoreType.DMA,
        ],
    )(table.reshape(B * n, D), idx.reshape(nidx))
    return out.reshape(B, nq, ns, D)


def _pad16(x):
    d = x.shape[-1]
    pad = (-d) % _L
    if pad:
        x = jnp.pad(x, [(0, 0)] * (x.ndim - 1) + [(0, pad)])
    return x


def _bn(x, g, b):
    axes = tuple(range(x.ndim - 1))
    mean = jnp.mean(x, axis=axes, keepdims=True)
    var = jnp.var(x, axis=axes, keepdims=True)
    return g * (x - mean) / jnp.sqrt(var + EPS) + b


def _linear(x, W, b=None):
    y = x @ W.T
    if b is not None:
        y = y + b
    return y


def _knn_body(k, pq_ref, pr_ref, idx_ref):
    _, Q, _ = pq_ref.shape
    _, _, nr = pr_ref.shape
    qx = pq_ref[0][:, 0:1]
    qy = pq_ref[0][:, 1:2]
    qz = pq_ref[0][:, 2:3]
    rx = pr_ref[0][0:1, :]
    ry = pr_ref[0][1:2, :]
    rz = pr_ref[0][2:3, :]
    d = (qx - rx) ** 2 + (qy - ry) ** 2 + (qz - rz) ** 2
    lane = jax.lax.broadcasted_iota(jnp.int32, (Q, nr), 1)
    js = []
    for _ in range(k):
        mn = jnp.min(d, axis=1, keepdims=True)
        j = jnp.min(jnp.where(d == mn, lane, nr), axis=1, keepdims=True)
        js.append(j)
        d = jnp.where(lane == j, jnp.inf, d)
    idx_ref[0] = jnp.concatenate(js, axis=1)


def _knn_pallas(pq, pr, k):
    B, nq, _ = pq.shape
    _, nr, _ = pr.shape
    Q = min(nq, 512)
    pr3 = jnp.transpose(pr, (0, 2, 1))
    return pl.pallas_call(
        functools.partial(_knn_body, k),
        grid=(B, nq // Q),
        in_specs=[
            pl.BlockSpec((1, Q, 3), lambda b, q: (b, q, 0)),
            pl.BlockSpec((1, 3, nr), lambda b, q: (b, 0, 0)),
        ],
        out_specs=pl.BlockSpec((1, Q, k), lambda b, q: (b, q, 0)),
        out_shape=jax.ShapeDtypeStruct((B, nq, k), jnp.int32),
    )(pq, pr3)


def _query_and_group_idx(idx, p_ref, p_query, feat, use_xyz):
    def per_batch(pr, pq, f, ib):
        g_xyz = pr[ib] - pq[:, None, :]
        g_feat = f[ib]
        if use_xyz:
            return jnp.concatenate([g_xyz, g_feat], axis=-1)
        return g_feat
    return jax.vmap(per_batch)(p_ref, p_query, feat, idx)


def _query_and_group(nsample, p_ref, p_query, feat, use_xyz):
    idx = _knn_pallas(p_query, p_ref, nsample)
    c = feat.shape[-1]
    tab = _pad16(jnp.concatenate([p_ref, feat], axis=-1))
    g = _sc_gather(tab, idx)
    g_xyz = g[..., :3] - p_query[:, :, None, :]
    g_feat = g[..., 3:3 + c]
    if use_xyz:
        return jnp.concatenate([g_xyz, g_feat], axis=-1)
    return g_feat


def _fps_body(m, p3_ref, idx_ref, np_ref, dists_ref):
    B, _, _, n8 = p3_ref.shape
    n = 8 * n8
    p3 = p3_ref[...]
    pos = (jax.lax.broadcasted_iota(jnp.int32, (B, 8, n8), 1) * n8
           + jax.lax.broadcasted_iota(jnp.int32, (B, 8, n8), 2))
    lane_m = jax.lax.broadcasted_iota(jnp.int32, (B, m), 1)
    lane_3m = jax.lax.broadcasted_iota(jnp.int32, (B, 3, m), 2)

    idx_ref[...] = jnp.zeros((B, m), jnp.int32)
    np_ref[...] = jnp.zeros((B, 3, m), jnp.float32)
    dists_ref[...] = jnp.full((B, 8, n8), jnp.inf, jnp.float32)

    def _rmax(a):
        return a.max(axis=2, keepdims=True).max(axis=1, keepdims=True)

    def _rmin(a):
        return a.min(axis=2, keepdims=True).min(axis=1, keepdims=True)

    def _rsum(a):
        return a.sum(axis=2, keepdims=True).sum(axis=1, keepdims=True)

    def _coords(last):
        sel = (pos == last).astype(jnp.float32)
        px = _rsum(p3[:, 0] * sel)
        py = _rsum(p3[:, 1] * sel)
        pz = _rsum(p3[:, 2] * sel)
        return px, py, pz

    def body(i, last):
        px, py, pz = _coords(last)
        d = (p3[:, 0] - px) ** 2 + (p3[:, 1] - py) ** 2 + (p3[:, 2] - pz) ** 2
        dists = jnp.minimum(dists_ref[...], d)
        dists_ref[...] = dists
        mx = _rmax(dists)
        nxt = _rmin(jnp.where(dists == mx, pos, n))
        idx_ref[...] = jnp.where(lane_m == i, nxt[:, :, 0], idx_ref[...])
        coords = jnp.concatenate([px, py, pz], axis=1)
        np_ref[...] = jnp.where(lane_3m == i - 1, coords, np_ref[...])
        return nxt

    last = jax.lax.fori_loop(1, m, body, jnp.zeros((B, 1, 1), jnp.int32))
    px, py, pz = _coords(last)
    coords = jnp.concatenate([px, py, pz], axis=1)
    np_ref[...] = jnp.where(lane_3m == m - 1, coords, np_ref[...])


def _fps_pallas(p, m):
    B, n, _ = p.shape
    n8 = n // 8
    p3 = jnp.transpose(p, (0, 2, 1)).reshape(B, 3, 8, n8)
    idx, np_ = pl.pallas_call(
        functools.partial(_fps_body, m),
        out_shape=(jax.ShapeDtypeStruct((B, m), jnp.int32),
                   jax.ShapeDtypeStruct((B, 3, m), jnp.float32)),
        scratch_shapes=[pltpu.VMEM((B, 8, n8), jnp.float32)],
    )(p3)
    return idx, jnp.transpose(np_, (0, 2, 1))


def _pt_layer(p, x, prm, nsample, share):
    xq = _linear(x, prm['q_w'], prm['q_b'])
    xk = _linear(x, prm['k_w'], prm['k_b'])
    xv = _linear(x, prm['v_w'], prm['v_b'])
    c = x.shape[-1]
    idx = _knn_pallas(p, p, nsample)
    tab = _pad16(jnp.concatenate([p, xk, xv], axis=-1))
    g = _sc_gather(tab, idx)
    p_r = g[..., :3] - p[:, :, None, :]
    xk_g = g[..., 3:3 + c]
    gv = g[..., 3 + c:3 + 2 * c]
    p_r = _linear(p_r, prm['p0_w'], prm['p0_b'])
    p_r = _bn(p_r, prm['p1_g'], prm['p1_b'])
    p_r = jax.nn.relu(p_r)
    p_r = _linear(p_r, prm['p3_w'], prm['p3_b'])
    w = xk_g - xq[:, :, None, :] + p_r
    w = _bn(w, prm['w0_g'], prm['w0_b'])
    w = jax.nn.relu(w)
    w = _linear(w, prm['w2_w'], prm['w2_b'])
    w = _bn(w, prm['w3_g'], prm['w3_b'])
    w = jax.nn.relu(w)
    w = _linear(w, prm['w5_w'], prm['w5_b'])
    w = jax.nn.softmax(w, axis=2)
    B, n, ns, c = gv.shape
    s = share
    out = ((gv + p_r).reshape(B, n, ns, s, c // s) * w[:, :, :, None, :]).sum(axis=2).reshape(B, n, c)
    return out


def _transition_down(p, x, prm, stride, nsample):
    if stride == 1:
        x = jax.nn.relu(_bn(_linear(x, prm['lin_w']), prm['bn_g'], prm['bn_b']))
        return p, x
    B, n, _ = p.shape
    m = n // stride
    idx, n_p = _fps_pallas(p, m)
    g = _query_and_group(nsample, p, n_p, x, True)
    y = _linear(g, prm['lin_w'])
    y = _bn(y, prm['bn_g'], prm['bn_b'])
    y = jax.nn.relu(y)
    y = jnp.max(y, axis=2)
    return n_p, y


def _pt_block(p, x, prm, nsample, share):
    identity = x
    x = jax.nn.relu(_bn(_linear(x, prm['l1_w']), prm['bn1_g'], prm['bn1_b']))
    x = jax.nn.relu(_bn(_pt_layer(p, x, prm['tr'], nsample, share), prm['bn2_g'], prm['bn2_b']))
    x = _bn(_linear(x, prm['l3_w']), prm['bn3_g'], prm['bn3_b'])
    x = jax.nn.relu(x + identity)
    return x


def _head_body(x_ref, w0_ref, b0_ref, w3_ref, b3_ref, out_ref):
    x = x_ref[...]
    g = jnp.mean(x, axis=1)
    h = g @ w0_ref[...].T + b0_ref[...]
    mean = jnp.mean(h, axis=0, keepdims=True)
    var = jnp.mean((h - mean) ** 2, axis=0, keepdims=True)
    h = (h - mean) / jnp.sqrt(var + EPS)
    h = jnp.maximum(h, 0.0)
    out_ref[...] = h @ w3_ref[...].T + b3_ref[...]


def _head(x, params):
    B, n, c = x.shape
    return pl.pallas_call(
        _head_body,
        out_shape=jax.ShapeDtypeStruct((B, c), jnp.float32),
    )(x, params['head0_w'], params['head0_b'], params['head3_w'], params['head3_b'])


def kernel(x_bcn, params):
    pts = jnp.transpose(x_bcn, (0, 2, 1))
    p = pts[:, :, :3]
    x = pts
    specs = [(1, 8), (4, 16), (4, 16), (4, 16)]
    for i, (stride, ns) in enumerate(specs):
        ep = params['enc%d' % (i + 1)]
        p, x = _transition_down(p, x, ep['td'], stride, ns)
        x = _pt_block(p, x, ep['blk'], ns, 8)
    return _head(x, params)
